# Initial kernel scaffold; baseline (speedup 1.0000x reference)
import jax
import jax.numpy as jnp
from jax.experimental import pallas as pl


def kernel(x, edge_index, W1_rel, W1_root, b1, W2_rel, W2_root, b2, W3_rel, W3_root, b3):
    raise NotImplementedError("write your pallas kernel here")


# speedup vs baseline: 5.8266x; 5.8266x over previous
import functools

import jax
import jax.numpy as jnp
from jax import lax
from jax.experimental import pallas as pl
from jax.experimental.pallas import tpu as pltpu
from jax.experimental.pallas import tpu_sc as plsc

N = 10000
E = 320000
D = 128
H = 128
HH = H // 2

NC = 2
NS = 16
NW = NC * NS
EPW = E // NW
K = 125
NCHUNK = EPW // K
NP = 10240
RPS = NP // NS
ZCH = 128
ROW_BLK = 2000


def _segment_sum_sc(h2, src3, dst3, zeros):
  mesh = plsc.VectorSubcoreMesh(core_axis_name="c", subcore_axis_name="s")

  @functools.partial(
      pl.kernel,
      out_type=jax.ShapeDtypeStruct((NC, 2, NP, HH), jnp.float32),
      mesh=mesh,
      compiler_params=pltpu.CompilerParams(use_tc_tiling_on_sc=False),
      scratch_types=[
          pltpu.VMEM((NCHUNK, K), jnp.int32),
          pltpu.VMEM((NCHUNK, K), jnp.int32),
          pltpu.VMEM((K, HH), jnp.float32),
          pltpu.VMEM((ZCH, HH), jnp.float32),
          pltpu.VMEM_SHARED((NP, HH), jnp.float32),
          pltpu.SemaphoreType.DMA,
      ],
  )
  def k(h_hbm, src_hbm, dst_hbm, z_hbm, out_hbm,
        src_v, dst_v, rows_v, stage_v, acc, sem):
    c = lax.axis_index("c")
    s = lax.axis_index("s")
    wid = c * NS + s

    pltpu.sync_copy(src_hbm.at[wid], src_v)
    pltpu.sync_copy(dst_hbm.at[wid], dst_v)
    pltpu.sync_copy(z_hbm, stage_v)

    for p in range(2):
      for j in range(RPS // ZCH):
        pltpu.sync_copy(stage_v, acc.at[pl.ds(s * RPS + j * ZCH, ZCH), :])
      plsc.subcore_barrier()

      def body(i, carry):
        pltpu.async_copy(h_hbm.at[p].at[src_v.at[i]], rows_v, sem).wait()
        pltpu.sync_copy(rows_v, acc.at[dst_v.at[i]], add=True)
        return carry

      lax.fori_loop(0, NCHUNK, body, 0)
      plsc.subcore_barrier()

      for j in range(RPS // ZCH):
        rows = pl.ds(s * RPS + j * ZCH, ZCH)
        pltpu.sync_copy(acc.at[rows, :], stage_v)
        pltpu.sync_copy(stage_v, out_hbm.at[c, p, rows, :])
      pltpu.sync_copy(z_hbm, stage_v)

  return k(h2, src3, dst3, zeros)


def _layer_tc(aggs, h2, w_rel_t, w_root_t, b2d, split_out):
  def body(aggs_ref, h_ref, wr_ref, wo_ref, b_ref, o_ref):
    f32 = jnp.float32
    y = jnp.dot(aggs_ref[0, 0] + aggs_ref[1, 0], wr_ref[:HH],
                preferred_element_type=f32)
    y += jnp.dot(aggs_ref[0, 1] + aggs_ref[1, 1], wr_ref[HH:],
                 preferred_element_type=f32)
    y += jnp.dot(h_ref[0], wo_ref[:HH], preferred_element_type=f32)
    y += jnp.dot(h_ref[1], wo_ref[HH:], preferred_element_type=f32)
    y = jnp.maximum(y + b_ref[...], 0.0)
    if split_out:
      o_ref[0] = y[:, :HH]
      o_ref[1] = y[:, HH:]
    else:
      o_ref[...] = y

  if split_out:
    out_shape = jax.ShapeDtypeStruct((2, N, HH), jnp.float32)
    out_spec = pl.BlockSpec((2, ROW_BLK, HH), lambda i: (0, i, 0))
  else:
    out_shape = jax.ShapeDtypeStruct((N, H), jnp.float32)
    out_spec = pl.BlockSpec((ROW_BLK, H), lambda i: (i, 0))

  return pl.pallas_call(
      body,
      grid=(N // ROW_BLK,),
      in_specs=[
          pl.BlockSpec((NC, 2, ROW_BLK, HH), lambda i: (0, 0, i, 0)),
          pl.BlockSpec((2, ROW_BLK, HH), lambda i: (0, i, 0)),
          pl.BlockSpec((H, H), lambda i: (0, 0)),
          pl.BlockSpec((H, H), lambda i: (0, 0)),
          pl.BlockSpec((1, H), lambda i: (0, 0)),
      ],
      out_specs=out_spec,
      out_shape=out_shape,
  )(aggs, h2, w_rel_t, w_root_t, b2d)


def kernel(x, edge_index, W1_rel, W1_root, b1, W2_rel, W2_root, b2,
           W3_rel, W3_root, b3):
  src3 = edge_index[0].reshape(NW, NCHUNK, K)
  dst3 = edge_index[1].reshape(NW, NCHUNK, K)
  zeros = jnp.zeros((ZCH, HH), jnp.float32)

  h2 = jnp.transpose(x.reshape(N, 2, HH), (1, 0, 2))
  layers = ((W1_rel, W1_root, b1),
            (W2_rel, W2_root, b2),
            (W3_rel, W3_root, b3))
  for li, (w_rel, w_root, b) in enumerate(layers):
    aggs = _segment_sum_sc(h2, src3, dst3, zeros)
    h2 = _layer_tc(aggs, h2, w_rel.T, w_root.T, b.reshape(1, H),
                   split_out=(li < 2))
  return h2.reshape(1, N, H)


# speedup vs baseline: 9.7547x; 1.6742x over previous
import functools

import jax
import jax.numpy as jnp
from jax import lax
from jax.experimental import pallas as pl
from jax.experimental.pallas import tpu as pltpu
from jax.experimental.pallas import tpu_sc as plsc

N = 10000
E = 320000
D = 128
H = 128
HH = H // 2

NC = 2
NS = 16
NW = NC * NS
EPW = E // NW
K = 125
NCHUNK = EPW // K
NP = 10240
RPS = NP // NS
ZCH = 128
ROW_BLK = 2000
NBUF = 4
AHEAD = 2


def _segment_sum_sc(h2, src3, dst3, zeros):
  mesh = plsc.VectorSubcoreMesh(core_axis_name="c", subcore_axis_name="s")

  @functools.partial(
      pl.kernel,
      out_type=jax.ShapeDtypeStruct((NC, 2, NP, HH), jnp.float32),
      mesh=mesh,
      compiler_params=pltpu.CompilerParams(use_tc_tiling_on_sc=False),
      scratch_types=[
          pltpu.VMEM((NCHUNK, K), jnp.int32),
          pltpu.VMEM((NCHUNK, K), jnp.int32),
          pltpu.VMEM((K, HH), jnp.float32),
          pltpu.VMEM((K, HH), jnp.float32),
          pltpu.VMEM((K, HH), jnp.float32),
          pltpu.VMEM((K, HH), jnp.float32),
          pltpu.VMEM((ZCH, HH), jnp.float32),
          pltpu.VMEM_SHARED((NP, HH), jnp.float32),
          pltpu.SemaphoreType.DMA,
          pltpu.SemaphoreType.DMA,
          pltpu.SemaphoreType.DMA,
          pltpu.SemaphoreType.DMA,
          pltpu.SemaphoreType.DMA,
          pltpu.SemaphoreType.DMA,
          pltpu.SemaphoreType.DMA,
          pltpu.SemaphoreType.DMA,
      ],
  )
  def k(h_hbm, src_hbm, dst_hbm, z_hbm, out_hbm,
        src_v, dst_v, rows0, rows1, rows2, rows3, stage_v, acc,
        gsem0, gsem1, gsem2, gsem3, ssem0, ssem1, ssem2, ssem3):
    rows = (rows0, rows1, rows2, rows3)
    gsem = (gsem0, gsem1, gsem2, gsem3)
    ssem = (ssem0, ssem1, ssem2, ssem3)
    c = lax.axis_index("c")
    s = lax.axis_index("s")
    wid = c * NS + s

    pltpu.sync_copy(src_hbm.at[wid], src_v)
    pltpu.sync_copy(dst_hbm.at[wid], dst_v)
    pltpu.sync_copy(z_hbm, stage_v)

    def gather(i, b):
      return pltpu.async_copy(h_hbm.at[p].at[src_v.at[i]], rows[b], gsem[b])

    def scatter(i, b):
      return pltpu.async_copy(rows[b], acc.at[dst_v.at[i]], ssem[b], add=True)

    for p in range(2):
      for j in range(RPS // ZCH):
        pltpu.sync_copy(stage_v, acc.at[pl.ds(s * RPS + j * ZCH, ZCH), :])
      plsc.subcore_barrier()

      gather(0, 0)
      gather(1, 1)

      def body(i, carry):
        for db in range(NBUF):
          idx = i + db
          bg = (db + AHEAD) % NBUF

          @pl.when(idx + AHEAD < NCHUNK)
          def _():
            @pl.when(idx >= AHEAD)
            def _():
              pltpu.make_async_copy(
                  rows[bg], acc.at[dst_v.at[idx - AHEAD]], ssem[bg]).wait()
            gather(idx + AHEAD, bg)

          pltpu.make_async_copy(
              h_hbm.at[p].at[src_v.at[idx]], rows[db], gsem[db]).wait()
          scatter(idx, db)
        return carry

      lax.fori_loop(0, NCHUNK // NBUF, lambda i, cr: body(i * NBUF, cr), 0,
                    unroll=False)
      for b in range(NBUF):
        pltpu.make_async_copy(
            rows[b], acc.at[dst_v.at[NCHUNK - NBUF + b]], ssem[b]).wait()
      plsc.subcore_barrier()

      for j in range(RPS // ZCH):
        rsl = pl.ds(s * RPS + j * ZCH, ZCH)
        pltpu.sync_copy(acc.at[rsl, :], stage_v)
        pltpu.sync_copy(stage_v, out_hbm.at[c, p, rsl, :])
      pltpu.sync_copy(z_hbm, stage_v)

  return k(h2, src3, dst3, zeros)


def _layer_tc(aggs, h2, w_rel_t, w_root_t, b2d, split_out):
  def body(aggs_ref, h_ref, wr_ref, wo_ref, b_ref, o_ref):
    f32 = jnp.float32
    y = jnp.dot(aggs_ref[0, 0] + aggs_ref[1, 0], wr_ref[:HH],
                preferred_element_type=f32)
    y += jnp.dot(aggs_ref[0, 1] + aggs_ref[1, 1], wr_ref[HH:],
                 preferred_element_type=f32)
    y += jnp.dot(h_ref[0], wo_ref[:HH], preferred_element_type=f32)
    y += jnp.dot(h_ref[1], wo_ref[HH:], preferred_element_type=f32)
    y = jnp.maximum(y + b_ref[...], 0.0)
    if split_out:
      o_ref[0] = y[:, :HH]
      o_ref[1] = y[:, HH:]
    else:
      o_ref[...] = y

  if split_out:
    out_shape = jax.ShapeDtypeStruct((2, N, HH), jnp.float32)
    out_spec = pl.BlockSpec((2, ROW_BLK, HH), lambda i: (0, i, 0))
  else:
    out_shape = jax.ShapeDtypeStruct((N, H), jnp.float32)
    out_spec = pl.BlockSpec((ROW_BLK, H), lambda i: (i, 0))

  return pl.pallas_call(
      body,
      grid=(N // ROW_BLK,),
      in_specs=[
          pl.BlockSpec((NC, 2, ROW_BLK, HH), lambda i: (0, 0, i, 0)),
          pl.BlockSpec((2, ROW_BLK, HH), lambda i: (0, i, 0)),
          pl.BlockSpec((H, H), lambda i: (0, 0)),
          pl.BlockSpec((H, H), lambda i: (0, 0)),
          pl.BlockSpec((1, H), lambda i: (0, 0)),
      ],
      out_specs=out_spec,
      out_shape=out_shape,
  )(aggs, h2, w_rel_t, w_root_t, b2d)


def kernel(x, edge_index, W1_rel, W1_root, b1, W2_rel, W2_root, b2,
           W3_rel, W3_root, b3):
  src3 = edge_index[0].reshape(NW, NCHUNK, K)
  dst3 = edge_index[1].reshape(NW, NCHUNK, K)
  zeros = jnp.zeros((ZCH, HH), jnp.float32)

  h2 = jnp.transpose(x.reshape(N, 2, HH), (1, 0, 2))
  layers = ((W1_rel, W1_root, b1),
            (W2_rel, W2_root, b2),
            (W3_rel, W3_root, b3))
  for li, (w_rel, w_root, b) in enumerate(layers):
    aggs = _segment_sum_sc(h2, src3, dst3, zeros)
    h2 = _layer_tc(aggs, h2, w_rel.T, w_root.T, b.reshape(1, H),
                   split_out=(li < 2))
  return h2.reshape(1, N, H)


# speedup vs baseline: 11.4268x; 1.1714x over previous
import functools

import jax
import jax.numpy as jnp
from jax import lax
from jax.experimental import pallas as pl
from jax.experimental.pallas import tpu as pltpu
from jax.experimental.pallas import tpu_sc as plsc

N = 10000
E = 320000
D = 128
H = 128
HH = H // 2

NC = 2
NS = 16
EPW = E // NS
K = 125
NCHUNK = EPW // K
NP = 10240
RPS = NP // NS
ZCH = 128
ROW_BLK = 2000
NBUF = 4
AHEAD = 2


def _segment_sum_sc(h2, src3, dst3, zeros):
  mesh = plsc.VectorSubcoreMesh(core_axis_name="c", subcore_axis_name="s")

  @functools.partial(
      pl.kernel,
      out_type=jax.ShapeDtypeStruct((NC, NP, HH), jnp.float32),
      mesh=mesh,
      compiler_params=pltpu.CompilerParams(use_tc_tiling_on_sc=False),
      scratch_types=[
          pltpu.VMEM((NCHUNK, K), jnp.int32),
          pltpu.VMEM((NCHUNK, K), jnp.int32),
          pltpu.VMEM((K, HH), jnp.float32),
          pltpu.VMEM((K, HH), jnp.float32),
          pltpu.VMEM((K, HH), jnp.float32),
          pltpu.VMEM((K, HH), jnp.float32),
          pltpu.VMEM((ZCH, HH), jnp.float32),
          pltpu.VMEM_SHARED((NP, HH), jnp.float32),
          pltpu.SemaphoreType.DMA,
          pltpu.SemaphoreType.DMA,
          pltpu.SemaphoreType.DMA,
          pltpu.SemaphoreType.DMA,
          pltpu.SemaphoreType.DMA,
          pltpu.SemaphoreType.DMA,
          pltpu.SemaphoreType.DMA,
          pltpu.SemaphoreType.DMA,
      ],
  )
  def k(h_hbm, src_hbm, dst_hbm, z_hbm, out_hbm,
        src_v, dst_v, rows0, rows1, rows2, rows3, stage_v, acc,
        gsem0, gsem1, gsem2, gsem3, ssem0, ssem1, ssem2, ssem3):
    rows = (rows0, rows1, rows2, rows3)
    gsem = (gsem0, gsem1, gsem2, gsem3)
    ssem = (ssem0, ssem1, ssem2, ssem3)
    c = lax.axis_index("c")
    s = lax.axis_index("s")
    hc = h_hbm.at[c]

    pltpu.sync_copy(src_hbm.at[s], src_v)
    pltpu.sync_copy(dst_hbm.at[s], dst_v)
    pltpu.sync_copy(z_hbm, stage_v)

    for j in range(RPS // ZCH):
      pltpu.sync_copy(stage_v, acc.at[pl.ds(s * RPS + j * ZCH, ZCH), :])
    plsc.subcore_barrier()

    def gather(i, b):
      return pltpu.async_copy(hc.at[src_v.at[i]], rows[b], gsem[b])

    def scatter(i, b):
      return pltpu.async_copy(rows[b], acc.at[dst_v.at[i]], ssem[b], add=True)

    gather(0, 0)
    gather(1, 1)

    def body(i, carry):
      for db in range(NBUF):
        idx = i + db
        bg = (db + AHEAD) % NBUF

        @pl.when(idx + AHEAD < NCHUNK)
        def _():
          @pl.when(idx >= AHEAD)
          def _():
            pltpu.make_async_copy(
                rows[bg], acc.at[dst_v.at[idx - AHEAD]], ssem[bg]).wait()
          gather(idx + AHEAD, bg)

        pltpu.make_async_copy(
            hc.at[src_v.at[idx]], rows[db], gsem[db]).wait()
        scatter(idx, db)
      return carry

    lax.fori_loop(0, NCHUNK // NBUF, lambda i, cr: body(i * NBUF, cr), 0,
                  unroll=False)
    for b in range(NBUF):
      pltpu.make_async_copy(
          rows[b], acc.at[dst_v.at[NCHUNK - NBUF + b]], ssem[b]).wait()
    plsc.subcore_barrier()

    for j in range(RPS // ZCH):
      rsl = pl.ds(s * RPS + j * ZCH, ZCH)
      pltpu.sync_copy(acc.at[rsl, :], stage_v)
      pltpu.sync_copy(stage_v, out_hbm.at[c, rsl, :])

  return k(h2, src3, dst3, zeros)


def _layer_tc(aggs, h2, w_rel_t, w_root_t, b2d, split_out):
  def body(aggs_ref, h_ref, wr_ref, wo_ref, b_ref, o_ref):
    f32 = jnp.float32
    y = jnp.dot(aggs_ref[0], wr_ref[:HH], preferred_element_type=f32)
    y += jnp.dot(aggs_ref[1], wr_ref[HH:], preferred_element_type=f32)
    y += jnp.dot(h_ref[0], wo_ref[:HH], preferred_element_type=f32)
    y += jnp.dot(h_ref[1], wo_ref[HH:], preferred_element_type=f32)
    y = jnp.maximum(y + b_ref[...], 0.0)
    if split_out:
      o_ref[0] = y[:, :HH]
      o_ref[1] = y[:, HH:]
    else:
      o_ref[...] = y

  if split_out:
    out_shape = jax.ShapeDtypeStruct((2, N, HH), jnp.float32)
    out_spec = pl.BlockSpec((2, ROW_BLK, HH), lambda i: (0, i, 0))
  else:
    out_shape = jax.ShapeDtypeStruct((N, H), jnp.float32)
    out_spec = pl.BlockSpec((ROW_BLK, H), lambda i: (i, 0))

  return pl.pallas_call(
      body,
      grid=(N // ROW_BLK,),
      in_specs=[
          pl.BlockSpec((NC, ROW_BLK, HH), lambda i: (0, i, 0)),
          pl.BlockSpec((2, ROW_BLK, HH), lambda i: (0, i, 0)),
          pl.BlockSpec((H, H), lambda i: (0, 0)),
          pl.BlockSpec((H, H), lambda i: (0, 0)),
          pl.BlockSpec((1, H), lambda i: (0, 0)),
      ],
      out_specs=out_spec,
      out_shape=out_shape,
  )(aggs, h2, w_rel_t, w_root_t, b2d)


def kernel(x, edge_index, W1_rel, W1_root, b1, W2_rel, W2_root, b2,
           W3_rel, W3_root, b3):
  src3 = edge_index[0].reshape(NS, NCHUNK, K)
  dst3 = edge_index[1].reshape(NS, NCHUNK, K)
  zeros = jnp.zeros((ZCH, HH), jnp.float32)

  h2 = jnp.transpose(x.reshape(N, 2, HH), (1, 0, 2))
  layers = ((W1_rel, W1_root, b1),
            (W2_rel, W2_root, b2),
            (W3_rel, W3_root, b3))
  for li, (w_rel, w_root, b) in enumerate(layers):
    aggs = _segment_sum_sc(h2, src3, dst3, zeros)
    h2 = _layer_tc(aggs, h2, w_rel.T, w_root.T, b.reshape(1, H),
                   split_out=(li < 2))
  return h2.reshape(1, N, H)


# speedup vs baseline: 12.0000x; 1.0502x over previous
import functools

import jax
import jax.numpy as jnp
from jax import lax
from jax.experimental import pallas as pl
from jax.experimental.pallas import tpu as pltpu
from jax.experimental.pallas import tpu_sc as plsc

N = 10000
E = 320000
D = 128
H = 128
HH = H // 2

NC = 2
NS = 16
EPW = E // NS
K = 125
NCHUNK = EPW // K
NP = 10240
RPS = NP // NS
ZCH = 128
ROW_BLK = 2000
NBUF = 4
AHEAD = 2


def _segment_sum_sc(h2, src3, dst3, zeros):
  mesh = plsc.VectorSubcoreMesh(core_axis_name="c", subcore_axis_name="s")

  @functools.partial(
      pl.kernel,
      out_type=jax.ShapeDtypeStruct((NC, NP, HH), jnp.float32),
      mesh=mesh,
      compiler_params=pltpu.CompilerParams(use_tc_tiling_on_sc=False),
      scratch_types=[
          pltpu.VMEM((NCHUNK, K), jnp.int32),
          pltpu.VMEM((NCHUNK, K), jnp.int32),
          pltpu.VMEM((K, HH), jnp.float32),
          pltpu.VMEM((K, HH), jnp.float32),
          pltpu.VMEM((K, HH), jnp.float32),
          pltpu.VMEM((K, HH), jnp.float32),
          pltpu.VMEM((ZCH, HH), jnp.float32),
          pltpu.VMEM_SHARED((NP, HH), jnp.float32),
          pltpu.SemaphoreType.DMA,
          pltpu.SemaphoreType.DMA,
          pltpu.SemaphoreType.DMA,
          pltpu.SemaphoreType.DMA,
          pltpu.SemaphoreType.DMA,
          pltpu.SemaphoreType.DMA,
          pltpu.SemaphoreType.DMA,
          pltpu.SemaphoreType.DMA,
      ],
  )
  def k(h_hbm, src_hbm, dst_hbm, z_hbm, out_hbm,
        src_v, dst_v, rows0, rows1, rows2, rows3, stage_v, acc,
        gsem0, gsem1, gsem2, gsem3, ssem0, ssem1, ssem2, ssem3):
    rows = (rows0, rows1, rows2, rows3)
    gsem = (gsem0, gsem1, gsem2, gsem3)
    ssem = (ssem0, ssem1, ssem2, ssem3)
    c = lax.axis_index("c")
    s = lax.axis_index("s")
    hc = h_hbm.at[c]

    pltpu.sync_copy(src_hbm.at[s], src_v)
    pltpu.sync_copy(dst_hbm.at[s], dst_v)
    pltpu.sync_copy(z_hbm, stage_v)

    for j in range(RPS // ZCH):
      pltpu.sync_copy(stage_v, acc.at[pl.ds(s * RPS + j * ZCH, ZCH), :])
    plsc.subcore_barrier()

    def gather(i, b):
      return pltpu.async_copy(hc.at[src_v.at[i]], rows[b], gsem[b])

    def scatter(i, b):
      return pltpu.async_copy(rows[b], acc.at[dst_v.at[i]], ssem[b], add=True)

    gather(0, 0)
    gather(1, 1)

    def body(i, carry):
      for db in range(NBUF):
        idx = i + db
        bg = (db + AHEAD) % NBUF

        @pl.when(idx + AHEAD < NCHUNK)
        def _():
          @pl.when(idx >= AHEAD)
          def _():
            pltpu.make_async_copy(
                rows[bg], acc.at[dst_v.at[idx - AHEAD]], ssem[bg]).wait()
          gather(idx + AHEAD, bg)

        pltpu.make_async_copy(
            hc.at[src_v.at[idx]], rows[db], gsem[db]).wait()
        scatter(idx, db)
      return carry

    lax.fori_loop(0, NCHUNK // NBUF, lambda i, cr: body(i * NBUF, cr), 0,
                  unroll=False)
    for b in range(NBUF):
      pltpu.make_async_copy(
          rows[b], acc.at[dst_v.at[NCHUNK - NBUF + b]], ssem[b]).wait()
    plsc.subcore_barrier()

    for j in range(RPS // ZCH):
      rsl = pl.ds(s * RPS + j * ZCH, ZCH)
      pltpu.sync_copy(acc.at[rsl, :], stage_v)
      pltpu.sync_copy(stage_v, out_hbm.at[c, rsl, :])

  return k(h2, src3, dst3, zeros)


def _layer_tc(aggs_i, h_i, mr, mo, b2, split_out):
  RB = ROW_BLK // 2

  def body(aggs_ref, h_ref, mr_ref, mo_ref, b_ref, o_ref):
    f32 = jnp.float32
    y = jnp.dot(aggs_ref[0], mr_ref[0], preferred_element_type=f32)
    y += jnp.dot(aggs_ref[1], mr_ref[1], preferred_element_type=f32)
    y += jnp.dot(h_ref[0], mo_ref[0], preferred_element_type=f32)
    y += jnp.dot(h_ref[1], mo_ref[1], preferred_element_type=f32)
    y = jnp.maximum(y + b_ref[...], 0.0)
    if split_out:
      o_ref[0] = jnp.concatenate([y[:, 0:HH], y[:, H:H + HH]], axis=1)
      o_ref[1] = jnp.concatenate([y[:, HH:H], y[:, H + HH:]], axis=1)
    else:
      o_ref[...] = y

  if split_out:
    out_shape = jax.ShapeDtypeStruct((2, N // 2, H), jnp.float32)
    out_spec = pl.BlockSpec((2, RB, H), lambda i: (0, i, 0))
  else:
    out_shape = jax.ShapeDtypeStruct((N // 2, 2 * H), jnp.float32)
    out_spec = pl.BlockSpec((RB, 2 * H), lambda i: (i, 0))

  return pl.pallas_call(
      body,
      grid=(N // ROW_BLK,),
      in_specs=[
          pl.BlockSpec((NC, RB, H), lambda i: (0, i, 0)),
          pl.BlockSpec((2, RB, H), lambda i: (0, i, 0)),
          pl.BlockSpec((2, H, 2 * H), lambda i: (0, 0, 0)),
          pl.BlockSpec((2, H, 2 * H), lambda i: (0, 0, 0)),
          pl.BlockSpec((1, 2 * H), lambda i: (0, 0)),
      ],
      out_specs=out_spec,
      out_shape=out_shape,
  )(aggs_i, h_i, mr, mo, b2)


def _expand_weight(w):
  wt = w.T
  z = jnp.zeros((HH, H), jnp.float32)
  m0 = jnp.concatenate(
      [jnp.concatenate([wt[:HH], z], axis=1),
       jnp.concatenate([z, wt[:HH]], axis=1)], axis=0)
  m1 = jnp.concatenate(
      [jnp.concatenate([wt[HH:], z], axis=1),
       jnp.concatenate([z, wt[HH:]], axis=1)], axis=0)
  return jnp.stack([m0, m1])


def kernel(x, edge_index, W1_rel, W1_root, b1, W2_rel, W2_root, b2,
           W3_rel, W3_root, b3):
  src3 = edge_index[0].reshape(NS, NCHUNK, K)
  dst3 = edge_index[1].reshape(NS, NCHUNK, K)
  zeros = jnp.zeros((ZCH, HH), jnp.float32)

  h_i = jnp.transpose(x.reshape(N // 2, 2, 2, HH), (2, 0, 1, 3))
  h_i = h_i.reshape(2, N // 2, H)
  layers = ((W1_rel, W1_root, b1),
            (W2_rel, W2_root, b2),
            (W3_rel, W3_root, b3))
  for li, (w_rel, w_root, b) in enumerate(layers):
    aggs = _segment_sum_sc(h_i.reshape(2, N, HH), src3, dst3, zeros)
    h_i = _layer_tc(aggs.reshape(NC, NP // 2, H), h_i,
                    _expand_weight(w_rel), _expand_weight(w_root),
                    jnp.concatenate([b, b]).reshape(1, 2 * H),
                    split_out=(li < 2))
  return h_i.reshape(1, N, H)


# speedup vs baseline: 13.1965x; 1.0997x over previous
import functools

import jax
import jax.numpy as jnp
from jax import lax
from jax.experimental import pallas as pl
from jax.experimental.pallas import tpu as pltpu
from jax.experimental.pallas import tpu_sc as plsc

N = 10000
E = 320000
D = 128
H = 128
HH = H // 2

NC = 2
NS = 16
EPW = E // NS
K = 125
NCHUNK = EPW // K
NP = 10240
RPS = NP // NS
ZCH = 128
ROW_BLK = 2000
NBUF = 4
AHEAD = 2


def _segment_sum_sc(h2, src3, dst3, zeros):
  mesh = plsc.VectorSubcoreMesh(core_axis_name="c", subcore_axis_name="s")

  @functools.partial(
      pl.kernel,
      out_type=jax.ShapeDtypeStruct((NC, NP, HH), jnp.float32),
      mesh=mesh,
      compiler_params=pltpu.CompilerParams(use_tc_tiling_on_sc=False),
      scratch_types=[
          pltpu.VMEM((NCHUNK, K), jnp.int32),
          pltpu.VMEM((NCHUNK, K), jnp.int32),
          pltpu.VMEM((K, HH), jnp.float32),
          pltpu.VMEM((K, HH), jnp.float32),
          pltpu.VMEM((K, HH), jnp.float32),
          pltpu.VMEM((K, HH), jnp.float32),
          pltpu.VMEM((ZCH, HH), jnp.float32),
          pltpu.VMEM_SHARED((NP, HH), jnp.float32),
          pltpu.SemaphoreType.DMA,
          pltpu.SemaphoreType.DMA,
          pltpu.SemaphoreType.DMA,
          pltpu.SemaphoreType.DMA,
          pltpu.SemaphoreType.DMA,
          pltpu.SemaphoreType.DMA,
          pltpu.SemaphoreType.DMA,
          pltpu.SemaphoreType.DMA,
      ],
  )
  def k(h_hbm, src_hbm, dst_hbm, z_hbm, out_hbm,
        src_v, dst_v, rows0, rows1, rows2, rows3, stage_v, acc,
        gsem0, gsem1, gsem2, gsem3, ssem0, ssem1, ssem2, ssem3):
    rows = (rows0, rows1, rows2, rows3)
    gsem = (gsem0, gsem1, gsem2, gsem3)
    ssem = (ssem0, ssem1, ssem2, ssem3)
    c = lax.axis_index("c")
    s = lax.axis_index("s")
    hc = h_hbm.at[c]

    pltpu.sync_copy(src_hbm.at[s], src_v)
    pltpu.sync_copy(dst_hbm.at[s], dst_v)
    pltpu.sync_copy(z_hbm, stage_v)

    for j in range(RPS // ZCH):
      pltpu.sync_copy(stage_v, acc.at[pl.ds(s * RPS + j * ZCH, ZCH), :])
    plsc.subcore_barrier()

    def gather(i, b):
      return pltpu.async_copy(hc.at[src_v.at[i]], rows[b], gsem[b])

    def scatter(i, b):
      return pltpu.async_copy(rows[b], acc.at[dst_v.at[i]], ssem[b], add=True)

    gather(0, 0)
    gather(1, 1)

    def body(i, carry):
      for db in range(NBUF):
        idx = i + db
        bg = (db + AHEAD) % NBUF

        @pl.when(idx + AHEAD < NCHUNK)
        def _():
          @pl.when(idx >= AHEAD)
          def _():
            pltpu.make_async_copy(
                rows[bg], acc.at[dst_v.at[idx - AHEAD]], ssem[bg]).wait()
          gather(idx + AHEAD, bg)

        pltpu.make_async_copy(
            hc.at[src_v.at[idx]], rows[db], gsem[db]).wait()
        scatter(idx, db)
      return carry

    lax.fori_loop(0, NCHUNK // NBUF, lambda i, cr: body(i * NBUF, cr), 0,
                  unroll=False)
    for b in range(NBUF):
      pltpu.make_async_copy(
          rows[b], acc.at[dst_v.at[NCHUNK - NBUF + b]], ssem[b]).wait()
    plsc.subcore_barrier()

    for j in range(RPS // ZCH):
      rsl = pl.ds(s * RPS + j * ZCH, ZCH)
      pltpu.sync_copy(acc.at[rsl, :], stage_v)
      pltpu.sync_copy(stage_v, out_hbm.at[c, rsl, :])

  return k(h2, src3, dst3, zeros)


def _layer_tc(aggs_i, h_i, wrt, wot, b2d, split_out):
  RB = ROW_BLK // 2

  def body(aggs_ref, h_ref, wr_ref, wo_ref, b_ref, o_ref):
    f32 = jnp.float32
    a0, a1, h0, h1 = aggs_ref[0], aggs_ref[1], h_ref[0], h_ref[1]

    def half(sl):
      y = jnp.dot(a0[:, sl], wr_ref[:HH], preferred_element_type=f32)
      y += jnp.dot(a1[:, sl], wr_ref[HH:], preferred_element_type=f32)
      y += jnp.dot(h0[:, sl], wo_ref[:HH], preferred_element_type=f32)
      y += jnp.dot(h1[:, sl], wo_ref[HH:], preferred_element_type=f32)
      return jnp.maximum(y + b_ref[...], 0.0)

    y_even = half(slice(0, HH))
    y_odd = half(slice(HH, H))
    if split_out:
      o_ref[0] = jnp.concatenate([y_even[:, :HH], y_odd[:, :HH]], axis=1)
      o_ref[1] = jnp.concatenate([y_even[:, HH:], y_odd[:, HH:]], axis=1)
    else:
      o_ref[...] = jnp.concatenate([y_even, y_odd], axis=1)

  if split_out:
    out_shape = jax.ShapeDtypeStruct((2, N // 2, H), jnp.float32)
    out_spec = pl.BlockSpec((2, RB, H), lambda i: (0, i, 0))
  else:
    out_shape = jax.ShapeDtypeStruct((N // 2, 2 * H), jnp.float32)
    out_spec = pl.BlockSpec((RB, 2 * H), lambda i: (i, 0))

  return pl.pallas_call(
      body,
      grid=(N // ROW_BLK,),
      in_specs=[
          pl.BlockSpec((NC, RB, H), lambda i: (0, i, 0)),
          pl.BlockSpec((2, RB, H), lambda i: (0, i, 0)),
          pl.BlockSpec((H, H), lambda i: (0, 0)),
          pl.BlockSpec((H, H), lambda i: (0, 0)),
          pl.BlockSpec((1, H), lambda i: (0, 0)),
      ],
      out_specs=out_spec,
      out_shape=out_shape,
  )(aggs_i, h_i, wrt, wot, b2d)


def _to_interleaved(x):
  RB = ROW_BLK // 2

  def body(x_ref, o_ref):
    v = x_ref[...].reshape(RB, 2, H)
    ev, od = v[:, 0, :], v[:, 1, :]
    o_ref[0] = jnp.concatenate([ev[:, :HH], od[:, :HH]], axis=1)
    o_ref[1] = jnp.concatenate([ev[:, HH:], od[:, HH:]], axis=1)

  return pl.pallas_call(
      body,
      grid=(N // ROW_BLK,),
      in_specs=[pl.BlockSpec((ROW_BLK, H), lambda i: (i, 0))],
      out_specs=pl.BlockSpec((2, RB, H), lambda i: (0, i, 0)),
      out_shape=jax.ShapeDtypeStruct((2, N // 2, H), jnp.float32),
  )(x)


def kernel(x, edge_index, W1_rel, W1_root, b1, W2_rel, W2_root, b2,
           W3_rel, W3_root, b3):
  src3 = edge_index[0].reshape(NS, NCHUNK, K)
  dst3 = edge_index[1].reshape(NS, NCHUNK, K)
  zeros = jnp.zeros((ZCH, HH), jnp.float32)

  h_i = _to_interleaved(x)
  layers = ((W1_rel, W1_root, b1),
            (W2_rel, W2_root, b2),
            (W3_rel, W3_root, b3))
  for li, (w_rel, w_root, b) in enumerate(layers):
    aggs = _segment_sum_sc(h_i.reshape(2, N, HH), src3, dst3, zeros)
    h_i = _layer_tc(aggs.reshape(NC, NP // 2, H), h_i,
                    w_rel.T, w_root.T, b.reshape(1, H),
                    split_out=(li < 2))
  return h_i.reshape(1, N, H)


# speedup vs baseline: 13.4180x; 1.0168x over previous
import functools

import jax
import jax.numpy as jnp
from jax import lax
from jax.experimental import pallas as pl
from jax.experimental.pallas import tpu as pltpu
from jax.experimental.pallas import tpu_sc as plsc

N = 10000
E = 320000
D = 128
H = 128
HH = H // 2

NC = 2
NS = 16
EPW = E // NS
K = 125
NCHUNK = EPW // K
NP = 10240
RPS = NP // NS
ZCH = 128
ROW_BLK = 2000
NBUF = 4
AHEAD = 2


def _segment_sum_sc(h2, e4, zeros):
  mesh = plsc.VectorSubcoreMesh(core_axis_name="c", subcore_axis_name="s")

  @functools.partial(
      pl.kernel,
      out_type=jax.ShapeDtypeStruct((NC, NP, HH), jnp.float32),
      mesh=mesh,
      compiler_params=pltpu.CompilerParams(use_tc_tiling_on_sc=False),
      scratch_types=[
          pltpu.VMEM((NCHUNK, K), jnp.int32),
          pltpu.VMEM((NCHUNK, K), jnp.int32),
          pltpu.VMEM((K, HH), jnp.float32),
          pltpu.VMEM((K, HH), jnp.float32),
          pltpu.VMEM((K, HH), jnp.float32),
          pltpu.VMEM((K, HH), jnp.float32),
          pltpu.VMEM((ZCH, HH), jnp.float32),
          pltpu.VMEM_SHARED((NP, HH), jnp.float32),
          pltpu.SemaphoreType.DMA,
          pltpu.SemaphoreType.DMA,
          pltpu.SemaphoreType.DMA,
          pltpu.SemaphoreType.DMA,
          pltpu.SemaphoreType.DMA,
          pltpu.SemaphoreType.DMA,
          pltpu.SemaphoreType.DMA,
          pltpu.SemaphoreType.DMA,
      ],
  )
  def k(h_hbm, e_hbm, z_hbm, out_hbm,
        src_v, dst_v, rows0, rows1, rows2, rows3, stage_v, acc,
        gsem0, gsem1, gsem2, gsem3, ssem0, ssem1, ssem2, ssem3):
    rows = (rows0, rows1, rows2, rows3)
    gsem = (gsem0, gsem1, gsem2, gsem3)
    ssem = (ssem0, ssem1, ssem2, ssem3)
    c = lax.axis_index("c")
    s = lax.axis_index("s")
    hc = h_hbm.at[c]

    pltpu.sync_copy(e_hbm.at[0, s], src_v)
    pltpu.sync_copy(e_hbm.at[1, s], dst_v)
    pltpu.sync_copy(z_hbm, stage_v)

    for j in range(RPS // ZCH):
      pltpu.sync_copy(stage_v, acc.at[pl.ds(s * RPS + j * ZCH, ZCH), :])
    plsc.subcore_barrier()

    def gather(i, b):
      return pltpu.async_copy(hc.at[src_v.at[i]], rows[b], gsem[b])

    def scatter(i, b):
      return pltpu.async_copy(rows[b], acc.at[dst_v.at[i]], ssem[b], add=True)

    gather(0, 0)
    gather(1, 1)

    def body(i, carry):
      for db in range(NBUF):
        idx = i + db
        bg = (db + AHEAD) % NBUF

        @pl.when(idx + AHEAD < NCHUNK)
        def _():
          @pl.when(idx >= AHEAD)
          def _():
            pltpu.make_async_copy(
                rows[bg], acc.at[dst_v.at[idx - AHEAD]], ssem[bg]).wait()
          gather(idx + AHEAD, bg)

        pltpu.make_async_copy(
            hc.at[src_v.at[idx]], rows[db], gsem[db]).wait()
        scatter(idx, db)
      return carry

    lax.fori_loop(0, NCHUNK // NBUF, lambda i, cr: body(i * NBUF, cr), 0,
                  unroll=False)
    for b in range(NBUF):
      pltpu.make_async_copy(
          rows[b], acc.at[dst_v.at[NCHUNK - NBUF + b]], ssem[b]).wait()
    plsc.subcore_barrier()

    for j in range(RPS // ZCH):
      rsl = pl.ds(s * RPS + j * ZCH, ZCH)
      pltpu.sync_copy(acc.at[rsl, :], stage_v)
      pltpu.sync_copy(stage_v, out_hbm.at[c, rsl, :])

  return k(h2, e4, zeros)


def _layer_tc(aggs_i, h_i, wrt, wot, b2d, split_out):
  RB = ROW_BLK // 2

  def body(aggs_ref, h_ref, wr_ref, wo_ref, b_ref, o_ref):
    f32 = jnp.float32
    a0, a1, h0, h1 = aggs_ref[0], aggs_ref[1], h_ref[0], h_ref[1]

    def half(sl):
      y = jnp.dot(a0[:, sl], wr_ref[:HH], preferred_element_type=f32)
      y += jnp.dot(a1[:, sl], wr_ref[HH:], preferred_element_type=f32)
      y += jnp.dot(h0[:, sl], wo_ref[:HH], preferred_element_type=f32)
      y += jnp.dot(h1[:, sl], wo_ref[HH:], preferred_element_type=f32)
      return jnp.maximum(y + b_ref[...], 0.0)

    y_even = half(slice(0, HH))
    y_odd = half(slice(HH, H))
    if split_out:
      o_ref[0] = jnp.concatenate([y_even[:, :HH], y_odd[:, :HH]], axis=1)
      o_ref[1] = jnp.concatenate([y_even[:, HH:], y_odd[:, HH:]], axis=1)
    else:
      o_ref[...] = jnp.concatenate([y_even, y_odd], axis=1)

  if split_out:
    out_shape = jax.ShapeDtypeStruct((2, N // 2, H), jnp.float32)
    out_spec = pl.BlockSpec((2, RB, H), lambda i: (0, i, 0))
  else:
    out_shape = jax.ShapeDtypeStruct((N // 2, 2 * H), jnp.float32)
    out_spec = pl.BlockSpec((RB, 2 * H), lambda i: (i, 0))

  return pl.pallas_call(
      body,
      grid=(N // ROW_BLK,),
      in_specs=[
          pl.BlockSpec((NC, RB, H), lambda i: (0, i, 0)),
          pl.BlockSpec((2, RB, H), lambda i: (0, i, 0)),
          pl.BlockSpec((H, H), lambda i: (0, 0)),
          pl.BlockSpec((H, H), lambda i: (0, 0)),
          pl.BlockSpec((1, H), lambda i: (0, 0)),
      ],
      out_specs=out_spec,
      out_shape=out_shape,
  )(aggs_i, h_i, wrt, wot, b2d)


def _edges_linear(edge_index):
  def body(e_ref, o_ref):
    o_ref[...] = e_ref[...].reshape(2, E // 128, 128)

  return pl.pallas_call(
      body,
      out_shape=jax.ShapeDtypeStruct((2, E // 128, 128), jnp.int32),
  )(edge_index)


def _to_interleaved(x):
  RB = ROW_BLK // 2

  def body(x_ref, o_ref):
    v = x_ref[...].reshape(RB, 2, H)
    ev, od = v[:, 0, :], v[:, 1, :]
    o_ref[0] = jnp.concatenate([ev[:, :HH], od[:, :HH]], axis=1)
    o_ref[1] = jnp.concatenate([ev[:, HH:], od[:, HH:]], axis=1)

  return pl.pallas_call(
      body,
      grid=(N // ROW_BLK,),
      in_specs=[pl.BlockSpec((ROW_BLK, H), lambda i: (i, 0))],
      out_specs=pl.BlockSpec((2, RB, H), lambda i: (0, i, 0)),
      out_shape=jax.ShapeDtypeStruct((2, N // 2, H), jnp.float32),
  )(x)


def kernel(x, edge_index, W1_rel, W1_root, b1, W2_rel, W2_root, b2,
           W3_rel, W3_root, b3):
  e4 = _edges_linear(edge_index).reshape(2, NS, NCHUNK, K)
  zeros = jnp.zeros((ZCH, HH), jnp.float32)

  h_i = _to_interleaved(x)
  layers = ((W1_rel, W1_root, b1),
            (W2_rel, W2_root, b2),
            (W3_rel, W3_root, b3))
  for li, (w_rel, w_root, b) in enumerate(layers):
    aggs = _segment_sum_sc(h_i.reshape(2, N, HH), e4, zeros)
    h_i = _layer_tc(aggs.reshape(NC, NP // 2, H), h_i,
                    w_rel.T, w_root.T, b.reshape(1, H),
                    split_out=(li < 2))
  return h_i.reshape(1, N, H)


# speedup vs baseline: 13.6194x; 1.0150x over previous
import functools

import jax
import jax.numpy as jnp
from jax import lax
from jax.experimental import pallas as pl
from jax.experimental.pallas import tpu as pltpu
from jax.experimental.pallas import tpu_sc as plsc

N = 10000
E = 320000
D = 128
H = 128
HH = H // 2

NC = 2
NS = 16
EPW = E // NS
K = 125
NCHUNK = EPW // K
NP = 10240
RPS = NP // NS
ZCH = 128
ROW_BLK = 2000
NBUF = 4
AHEAD = 2


def _segment_sum_sc(h2, e4, zeros):
  mesh = plsc.VectorSubcoreMesh(core_axis_name="c", subcore_axis_name="s")

  @functools.partial(
      pl.kernel,
      out_type=jax.ShapeDtypeStruct((NC, NP, HH), jnp.float32),
      mesh=mesh,
      compiler_params=pltpu.CompilerParams(use_tc_tiling_on_sc=False),
      scratch_types=[
          pltpu.VMEM((NCHUNK, K), jnp.int32),
          pltpu.VMEM((NCHUNK, K), jnp.int32),
          pltpu.VMEM((K, HH), jnp.float32),
          pltpu.VMEM((K, HH), jnp.float32),
          pltpu.VMEM((K, HH), jnp.float32),
          pltpu.VMEM((K, HH), jnp.float32),
          pltpu.VMEM((ZCH, HH), jnp.float32),
          pltpu.VMEM_SHARED((NP, HH), jnp.float32),
          pltpu.SemaphoreType.DMA,
          pltpu.SemaphoreType.DMA,
          pltpu.SemaphoreType.DMA,
          pltpu.SemaphoreType.DMA,
          pltpu.SemaphoreType.DMA,
          pltpu.SemaphoreType.DMA,
          pltpu.SemaphoreType.DMA,
          pltpu.SemaphoreType.DMA,
      ],
  )
  def k(h_hbm, e_hbm, z_hbm, out_hbm,
        src_v, dst_v, rows0, rows1, rows2, rows3, stage_v, acc,
        gsem0, gsem1, gsem2, gsem3, ssem0, ssem1, ssem2, ssem3):
    rows = (rows0, rows1, rows2, rows3)
    gsem = (gsem0, gsem1, gsem2, gsem3)
    ssem = (ssem0, ssem1, ssem2, ssem3)
    c = lax.axis_index("c")
    s = lax.axis_index("s")
    hc = h_hbm.at[c]

    pltpu.sync_copy(e_hbm.at[0, s], src_v)
    pltpu.sync_copy(e_hbm.at[1, s], dst_v)
    pltpu.sync_copy(z_hbm, stage_v)

    for j in range(RPS // ZCH):
      pltpu.sync_copy(stage_v, acc.at[pl.ds(s * RPS + j * ZCH, ZCH), :])
    plsc.subcore_barrier()

    def gather(i, b):
      return pltpu.async_copy(hc.at[src_v.at[i]], rows[b], gsem[b])

    def scatter(i, b):
      return pltpu.async_copy(rows[b], acc.at[dst_v.at[i]], ssem[b], add=True)

    gather(0, 0)
    gather(1, 1)

    def body(i, carry):
      for db in range(NBUF):
        idx = i + db
        bg = (db + AHEAD) % NBUF

        @pl.when(idx + AHEAD < NCHUNK)
        def _():
          @pl.when(idx >= AHEAD)
          def _():
            pltpu.make_async_copy(
                rows[bg], acc.at[dst_v.at[idx - AHEAD]], ssem[bg]).wait()
          gather(idx + AHEAD, bg)

        pltpu.make_async_copy(
            hc.at[src_v.at[idx]], rows[db], gsem[db]).wait()
        scatter(idx, db)
      return carry

    lax.fori_loop(0, NCHUNK // NBUF, lambda i, cr: body(i * NBUF, cr), 0,
                  unroll=False)
    for b in range(NBUF):
      pltpu.make_async_copy(
          rows[b], acc.at[dst_v.at[NCHUNK - NBUF + b]], ssem[b]).wait()
    plsc.subcore_barrier()

    for j in range(RPS // ZCH):
      rsl = pl.ds(s * RPS + j * ZCH, ZCH)
      pltpu.sync_copy(acc.at[rsl, :], stage_v)
      pltpu.sync_copy(stage_v, out_hbm.at[c, rsl, :])

  return k(h2, e4, zeros)


def _root_tc(h_i, wot, b2d):
  RB = ROW_BLK // 2

  def body(h_ref, wo_ref, b_ref, o_ref):
    f32 = jnp.float32
    h0, h1 = h_ref[0], h_ref[1]

    def half(sl):
      y = jnp.dot(h0[:, sl], wo_ref[:HH], preferred_element_type=f32)
      y += jnp.dot(h1[:, sl], wo_ref[HH:], preferred_element_type=f32)
      return y + b_ref[...]

    o_ref[...] = jnp.concatenate([half(slice(0, HH)), half(slice(HH, H))],
                                 axis=1)

  return pl.pallas_call(
      body,
      grid=(N // ROW_BLK,),
      in_specs=[
          pl.BlockSpec((2, RB, H), lambda i: (0, i, 0)),
          pl.BlockSpec((H, H), lambda i: (0, 0)),
          pl.BlockSpec((1, H), lambda i: (0, 0)),
      ],
      out_specs=pl.BlockSpec((RB, 2 * H), lambda i: (i, 0)),
      out_shape=jax.ShapeDtypeStruct((N // 2, 2 * H), jnp.float32),
  )(h_i, wot, b2d)


def _combine_tc(aggs_i, r_pair, wrt, split_out):
  RB = ROW_BLK // 2

  def body(aggs_ref, r_ref, wr_ref, o_ref):
    f32 = jnp.float32
    a0, a1 = aggs_ref[0], aggs_ref[1]

    def half(sl, rsl):
      y = jnp.dot(a0[:, sl], wr_ref[:HH], preferred_element_type=f32)
      y += jnp.dot(a1[:, sl], wr_ref[HH:], preferred_element_type=f32)
      return jnp.maximum(y + r_ref[:, rsl], 0.0)

    y_even = half(slice(0, HH), slice(0, H))
    y_odd = half(slice(HH, H), slice(H, 2 * H))
    if split_out:
      o_ref[0] = jnp.concatenate([y_even[:, :HH], y_odd[:, :HH]], axis=1)
      o_ref[1] = jnp.concatenate([y_even[:, HH:], y_odd[:, HH:]], axis=1)
    else:
      v = jnp.concatenate([y_even[:, None, :], y_odd[:, None, :]], axis=1)
      o_ref[...] = v.reshape(ROW_BLK, H)

  if split_out:
    out_shape = jax.ShapeDtypeStruct((2, N // 2, H), jnp.float32)
    out_spec = pl.BlockSpec((2, RB, H), lambda i: (0, i, 0))
  else:
    out_shape = jax.ShapeDtypeStruct((N, H), jnp.float32)
    out_spec = pl.BlockSpec((ROW_BLK, H), lambda i: (i, 0))

  return pl.pallas_call(
      body,
      grid=(N // ROW_BLK,),
      in_specs=[
          pl.BlockSpec((NC, RB, H), lambda i: (0, i, 0)),
          pl.BlockSpec((RB, 2 * H), lambda i: (i, 0)),
          pl.BlockSpec((H, H), lambda i: (0, 0)),
      ],
      out_specs=out_spec,
      out_shape=out_shape,
  )(aggs_i, r_pair, wrt)


def _edges_linear(edge_index):
  def body(e_ref, o_ref):
    o_ref[...] = e_ref[...].reshape(2, E // 128, 128)

  return pl.pallas_call(
      body,
      out_shape=jax.ShapeDtypeStruct((2, E // 128, 128), jnp.int32),
  )(edge_index)


def _to_interleaved(x):
  RB = ROW_BLK // 2

  def body(x_ref, o_ref):
    v = x_ref[...].reshape(RB, 2, H)
    ev, od = v[:, 0, :], v[:, 1, :]
    o_ref[0] = jnp.concatenate([ev[:, :HH], od[:, :HH]], axis=1)
    o_ref[1] = jnp.concatenate([ev[:, HH:], od[:, HH:]], axis=1)

  return pl.pallas_call(
      body,
      grid=(N // ROW_BLK,),
      in_specs=[pl.BlockSpec((ROW_BLK, H), lambda i: (i, 0))],
      out_specs=pl.BlockSpec((2, RB, H), lambda i: (0, i, 0)),
      out_shape=jax.ShapeDtypeStruct((2, N // 2, H), jnp.float32),
  )(x)


def kernel(x, edge_index, W1_rel, W1_root, b1, W2_rel, W2_root, b2,
           W3_rel, W3_root, b3):
  e4 = _edges_linear(edge_index).reshape(2, NS, NCHUNK, K)
  zeros = jnp.zeros((ZCH, HH), jnp.float32)

  h_i = _to_interleaved(x)
  layers = ((W1_rel, W1_root, b1),
            (W2_rel, W2_root, b2),
            (W3_rel, W3_root, b3))
  for li, (w_rel, w_root, b) in enumerate(layers):
    aggs = _segment_sum_sc(h_i.reshape(2, N, HH), e4, zeros)
    r_pair = _root_tc(h_i, w_root.T, b.reshape(1, H))
    h_i = _combine_tc(aggs.reshape(NC, NP // 2, H), r_pair, w_rel.T,
                      split_out=(li < 2))
  return h_i.reshape(1, N, H)


# speedup vs baseline: 13.8633x; 1.0179x over previous
import functools

import jax
import jax.numpy as jnp
from jax import lax
from jax.experimental import pallas as pl
from jax.experimental.pallas import tpu as pltpu
from jax.experimental.pallas import tpu_sc as plsc

N = 10000
E = 320000
D = 128
H = 128
HH = H // 2

NC = 2
NS = 16
EPW = E // NS
K = 125
NCHUNK = EPW // K
NP = 10240
RPS = NP // NS
ZCH = 128
ROW_BLK = 2000
NBUF = 4
AHEAD = 2


def _segment_sum_sc(h2, e4, zeros):
  mesh = plsc.VectorSubcoreMesh(core_axis_name="c", subcore_axis_name="s")

  @functools.partial(
      pl.kernel,
      out_type=jax.ShapeDtypeStruct((NC, NP, HH), jnp.float32),
      mesh=mesh,
      compiler_params=pltpu.CompilerParams(use_tc_tiling_on_sc=False),
      scratch_types=[
          pltpu.VMEM((NCHUNK, K), jnp.int32),
          pltpu.VMEM((NCHUNK, K), jnp.int32),
          pltpu.VMEM((K, HH), jnp.float32),
          pltpu.VMEM((K, HH), jnp.float32),
          pltpu.VMEM((K, HH), jnp.float32),
          pltpu.VMEM((K, HH), jnp.float32),
          pltpu.VMEM((ZCH, HH), jnp.float32),
          pltpu.VMEM_SHARED((NP, HH), jnp.float32),
          pltpu.SemaphoreType.DMA,
          pltpu.SemaphoreType.DMA,
          pltpu.SemaphoreType.DMA,
          pltpu.SemaphoreType.DMA,
          pltpu.SemaphoreType.DMA,
          pltpu.SemaphoreType.DMA,
          pltpu.SemaphoreType.DMA,
          pltpu.SemaphoreType.DMA,
      ],
  )
  def k(h_hbm, e_hbm, z_hbm, out_hbm,
        src_v, dst_v, rows0, rows1, rows2, rows3, stage_v, acc,
        gsem0, gsem1, gsem2, gsem3, ssem0, ssem1, ssem2, ssem3):
    rows = (rows0, rows1, rows2, rows3)
    gsem = (gsem0, gsem1, gsem2, gsem3)
    ssem = (ssem0, ssem1, ssem2, ssem3)
    c = lax.axis_index("c")
    s = lax.axis_index("s")
    hc = h_hbm.at[c]

    cp_src = pltpu.async_copy(e_hbm.at[0, s], src_v, gsem0)
    cp_dst = pltpu.async_copy(e_hbm.at[1, s], dst_v, gsem1)
    pltpu.sync_copy(z_hbm, stage_v)
    zcp = [pltpu.async_copy(stage_v, acc.at[pl.ds(s * RPS + j * ZCH, ZCH), :],
                            ssem0) for j in range(RPS // ZCH)]
    cp_src.wait()
    cp_dst.wait()
    for cp in zcp:
      cp.wait()
    plsc.subcore_barrier()

    def gather(i, b):
      return pltpu.async_copy(hc.at[src_v.at[i]], rows[b], gsem[b])

    def scatter(i, b):
      return pltpu.async_copy(rows[b], acc.at[dst_v.at[i]], ssem[b], add=True)

    gather(0, 0)
    gather(1, 1)

    def body(i, carry):
      for db in range(NBUF):
        idx = i + db
        bg = (db + AHEAD) % NBUF

        @pl.when(idx + AHEAD < NCHUNK)
        def _():
          @pl.when(idx >= AHEAD)
          def _():
            pltpu.make_async_copy(
                rows[bg], acc.at[dst_v.at[idx - AHEAD]], ssem[bg]).wait()
          gather(idx + AHEAD, bg)

        pltpu.make_async_copy(
            hc.at[src_v.at[idx]], rows[db], gsem[db]).wait()
        scatter(idx, db)
      return carry

    lax.fori_loop(0, NCHUNK // NBUF, lambda i, cr: body(i * NBUF, cr), 0,
                  unroll=False)
    for b in range(NBUF):
      pltpu.make_async_copy(
          rows[b], acc.at[dst_v.at[NCHUNK - NBUF + b]], ssem[b]).wait()
    plsc.subcore_barrier()

    ocp = []
    for j in range(RPS // ZCH):
      rsl = pl.ds(s * RPS + j * ZCH, ZCH)
      ocp.append(pltpu.async_copy(acc.at[rsl, :], out_hbm.at[c, rsl, :],
                                  gsem[j % NBUF]))
    for cp in ocp:
      cp.wait()

  return k(h2, e4, zeros)


def _root_tc(h_i, wot, b2d):
  RB = ROW_BLK // 2

  def body(h_ref, wo_ref, b_ref, o_ref):
    f32 = jnp.float32
    h0, h1 = h_ref[0], h_ref[1]

    def half(sl):
      y = jnp.dot(h0[:, sl], wo_ref[:HH], preferred_element_type=f32)
      y += jnp.dot(h1[:, sl], wo_ref[HH:], preferred_element_type=f32)
      return y + b_ref[...]

    o_ref[...] = jnp.concatenate([half(slice(0, HH)), half(slice(HH, H))],
                                 axis=1)

  return pl.pallas_call(
      body,
      grid=(N // ROW_BLK,),
      in_specs=[
          pl.BlockSpec((2, RB, H), lambda i: (0, i, 0)),
          pl.BlockSpec((H, H), lambda i: (0, 0)),
          pl.BlockSpec((1, H), lambda i: (0, 0)),
      ],
      out_specs=pl.BlockSpec((RB, 2 * H), lambda i: (i, 0)),
      out_shape=jax.ShapeDtypeStruct((N // 2, 2 * H), jnp.float32),
  )(h_i, wot, b2d)


def _combine_tc(aggs_i, r_pair, wrt, split_out):
  RB = ROW_BLK // 2

  def body(aggs_ref, r_ref, wr_ref, o_ref):
    f32 = jnp.float32
    a0, a1 = aggs_ref[0], aggs_ref[1]

    def half(sl, rsl):
      y = jnp.dot(a0[:, sl], wr_ref[:HH], preferred_element_type=f32)
      y += jnp.dot(a1[:, sl], wr_ref[HH:], preferred_element_type=f32)
      return jnp.maximum(y + r_ref[:, rsl], 0.0)

    y_even = half(slice(0, HH), slice(0, H))
    y_odd = half(slice(HH, H), slice(H, 2 * H))
    if split_out:
      o_ref[0] = jnp.concatenate([y_even[:, :HH], y_odd[:, :HH]], axis=1)
      o_ref[1] = jnp.concatenate([y_even[:, HH:], y_odd[:, HH:]], axis=1)
    else:
      v = jnp.concatenate([y_even[:, None, :], y_odd[:, None, :]], axis=1)
      o_ref[...] = v.reshape(ROW_BLK, H)

  if split_out:
    out_shape = jax.ShapeDtypeStruct((2, N // 2, H), jnp.float32)
    out_spec = pl.BlockSpec((2, RB, H), lambda i: (0, i, 0))
  else:
    out_shape = jax.ShapeDtypeStruct((N, H), jnp.float32)
    out_spec = pl.BlockSpec((ROW_BLK, H), lambda i: (i, 0))

  return pl.pallas_call(
      body,
      grid=(N // ROW_BLK,),
      in_specs=[
          pl.BlockSpec((NC, RB, H), lambda i: (0, i, 0)),
          pl.BlockSpec((RB, 2 * H), lambda i: (i, 0)),
          pl.BlockSpec((H, H), lambda i: (0, 0)),
      ],
      out_specs=out_spec,
      out_shape=out_shape,
  )(aggs_i, r_pair, wrt)


def _edges_linear(edge_index):
  def body(e_ref, o_ref):
    o_ref[...] = e_ref[...].reshape(2, E // 128, 128)

  return pl.pallas_call(
      body,
      out_shape=jax.ShapeDtypeStruct((2, E // 128, 128), jnp.int32),
  )(edge_index)


def _to_interleaved(x):
  RB = ROW_BLK // 2

  def body(x_ref, o_ref):
    v = x_ref[...].reshape(RB, 2, H)
    ev, od = v[:, 0, :], v[:, 1, :]
    o_ref[0] = jnp.concatenate([ev[:, :HH], od[:, :HH]], axis=1)
    o_ref[1] = jnp.concatenate([ev[:, HH:], od[:, HH:]], axis=1)

  return pl.pallas_call(
      body,
      grid=(N // ROW_BLK,),
      in_specs=[pl.BlockSpec((ROW_BLK, H), lambda i: (i, 0))],
      out_specs=pl.BlockSpec((2, RB, H), lambda i: (0, i, 0)),
      out_shape=jax.ShapeDtypeStruct((2, N // 2, H), jnp.float32),
  )(x)


def kernel(x, edge_index, W1_rel, W1_root, b1, W2_rel, W2_root, b2,
           W3_rel, W3_root, b3):
  e4 = _edges_linear(edge_index).reshape(2, NS, NCHUNK, K)
  zeros = jnp.zeros((ZCH, HH), jnp.float32)

  h_i = _to_interleaved(x)
  layers = ((W1_rel, W1_root, b1),
            (W2_rel, W2_root, b2),
            (W3_rel, W3_root, b3))
  for li, (w_rel, w_root, b) in enumerate(layers):
    aggs = _segment_sum_sc(h_i.reshape(2, N, HH), e4, zeros)
    r_pair = _root_tc(h_i, w_root.T, b.reshape(1, H))
    h_i = _combine_tc(aggs.reshape(NC, NP // 2, H), r_pair, w_rel.T,
                      split_out=(li < 2))
  return h_i.reshape(1, N, H)
